# baseline (speedup 1.0000x reference)
import jax
import jax.numpy as jnp
from jax import lax
from jax.experimental import pallas as pl
from jax.experimental.pallas import tpu as pltpu

ROWS = 128
COLS = 100000
RB = 8
TW = 2048
NT = (COLS + TW - 1) // TW
NR = ROWS // RB

_R0 = (13, 15, 26, 6)
_R1 = (17, 29, 16, 24)
_KS0 = 0
_KS1 = 42
_KS2 = _KS0 ^ _KS1 ^ 0x1BD11BDA


def _round_group(x0, x1, rots):
    for r in rots:
        x0 = x0 + x1
        x1 = ((x1 << jnp.uint32(r)) | (x1 >> jnp.uint32(32 - r))) ^ x0
    return x0, x1


def _threefry_bits(n):
    ks0 = jnp.uint32(_KS0)
    ks1 = jnp.uint32(_KS1)
    ks2 = jnp.uint32(_KS2)
    x0 = jnp.zeros_like(n)
    x1 = n + ks1
    x0, x1 = _round_group(x0, x1, _R0)
    x0 = x0 + ks1
    x1 = x1 + jnp.uint32(_KS2 + 1)
    x0, x1 = _round_group(x0, x1, _R1)
    x0 = x0 + ks2
    x1 = x1 + jnp.uint32(_KS0 + 2)
    x0, x1 = _round_group(x0, x1, _R0)
    x0 = x0 + ks0
    x1 = x1 + jnp.uint32(_KS1 + 3)
    x0, x1 = _round_group(x0, x1, _R1)
    x0 = x0 + ks1
    x1 = x1 + jnp.uint32(_KS2 + 4)
    x0, x1 = _round_group(x0, x1, _R0)
    x0 = x0 + ks2
    x1 = x1 + jnp.uint32(_KS0 + 5)
    return x0 ^ x1


def _kernel(logits_ref, out_ref, e_scr, bits_scr, s_scr, r_scr):
    rb = pl.program_id(0)
    ph = pl.program_id(1)
    c = pl.program_id(2)

    @pl.when(ph == 0)
    def _phase0():
        @pl.when(c == 0)
        def _init():
            s_scr[...] = jnp.zeros((RB, TW), jnp.float32)

        @pl.when(c > 0)
        def _tail():
            t = c - 1
            bits = bits_scr[t % 2]
            fb = (bits >> jnp.uint32(9)) | jnp.uint32(0x3F800000)
            u = lax.bitcast_convert_type(fb, jnp.float32) - jnp.float32(1.0)
            eps = jnp.float32(1e-10)
            g = -jnp.log(eps - jnp.log(u + eps))
            z = logits_ref[...] + g
            e = jnp.exp(z)
            col = t * TW + lax.broadcasted_iota(jnp.int32, (RB, TW), 1)
            e = jnp.where(col < COLS, e, jnp.float32(0.0))
            e_scr[t] = e
            s_scr[...] += e

        @pl.when(c < NT)
        def _hash():
            row = rb * RB + lax.broadcasted_iota(jnp.int32, (RB, TW), 0)
            col = c * TW + lax.broadcasted_iota(jnp.int32, (RB, TW), 1)
            n = (row * COLS + col).astype(jnp.uint32)
            bits_scr[c % 2] = _threefry_bits(n)

    @pl.when(ph == 1)
    def _phase1():
        @pl.when(c == 0)
        def _recip():
            s = jnp.sum(s_scr[...], axis=1, keepdims=True)
            r_scr[...] = jnp.broadcast_to(jnp.float32(1.0) / s, (RB, TW))

        @pl.when(c < NT)
        def _scale():
            out_ref[...] = e_scr[c] * r_scr[...]


def kernel(logits):
    def logits_idx(r, ph, c):
        t = jnp.where(c > 0, c - 1, 0)
        return (r, jnp.where(ph == 0, t, 0))

    def out_idx(r, ph, c):
        return (r, jnp.where(ph == 1, jnp.minimum(c, NT - 1), 0))

    return pl.pallas_call(
        _kernel,
        grid=(NR, 2, NT + 1),
        in_specs=[pl.BlockSpec((RB, TW), logits_idx)],
        out_specs=pl.BlockSpec((RB, TW), out_idx),
        out_shape=jax.ShapeDtypeStruct((ROWS, COLS), jnp.float32),
        scratch_shapes=[
            pltpu.VMEM((NT, RB, TW), jnp.float32),
            pltpu.VMEM((2, RB, TW), jnp.uint32),
            pltpu.VMEM((RB, TW), jnp.float32),
            pltpu.VMEM((RB, TW), jnp.float32),
        ],
    )(logits)


# speedup vs baseline: 1.3537x; 1.3537x over previous
import jax
import jax.numpy as jnp
from jax import lax
from jax.experimental import pallas as pl
from jax.experimental.pallas import tpu as pltpu

ROWS = 128
COLS = 100000
RB = 8
TW = 2048
NT = (COLS + TW - 1) // TW
NR = ROWS // RB

_R0 = (13, 15, 26, 6)
_R1 = (17, 29, 16, 24)
_KS0 = 0
_KS1 = 42
_KS2 = _KS0 ^ _KS1 ^ 0x1BD11BDA


def _round_group(x0, x1, rots):
    for r in rots:
        x0 = x0 + x1
        x1 = ((x1 << jnp.uint32(r)) | (x1 >> jnp.uint32(32 - r))) ^ x0
    return x0, x1


def _threefry_bits(n):
    ks0 = jnp.uint32(_KS0)
    ks1 = jnp.uint32(_KS1)
    ks2 = jnp.uint32(_KS2)
    x0 = jnp.zeros_like(n)
    x1 = n + ks1
    x0, x1 = _round_group(x0, x1, _R0)
    x0 = x0 + ks1
    x1 = x1 + jnp.uint32(_KS2 + 1)
    x0, x1 = _round_group(x0, x1, _R1)
    x0 = x0 + ks2
    x1 = x1 + jnp.uint32(_KS0 + 2)
    x0, x1 = _round_group(x0, x1, _R0)
    x0 = x0 + ks0
    x1 = x1 + jnp.uint32(_KS1 + 3)
    x0, x1 = _round_group(x0, x1, _R1)
    x0 = x0 + ks1
    x1 = x1 + jnp.uint32(_KS2 + 4)
    x0, x1 = _round_group(x0, x1, _R0)
    x0 = x0 + ks2
    x1 = x1 + jnp.uint32(_KS0 + 5)
    return x0 ^ x1


def _kernel(logits_ref, out_ref, e_scr, bits_scr, s_scr, r_scr, m_scr):
    r = pl.program_id(0)
    c = pl.program_id(1)

    @pl.when((r == 0) & (c == 0))
    def _mask_init():
        col = (NT - 1) * TW + lax.broadcasted_iota(jnp.int32, (RB, TW), 1)
        m_scr[0] = jnp.ones((RB, TW), jnp.float32)
        m_scr[1] = jnp.where(col < COLS, jnp.float32(1.0), jnp.float32(0.0))

    @pl.when((r > 0) & (c == 0))
    def _recip():
        s = jnp.sum(s_scr[(r - 1) % 2], axis=1, keepdims=True)
        r_scr[...] = jnp.broadcast_to(jnp.float32(1.0) / s, (RB, TW))

    @pl.when((r < NR) & (c == 0))
    def _s_init():
        s_scr[r % 2] = jnp.zeros((RB, TW), jnp.float32)

    @pl.when((r < NR) & (c > 0))
    def _tail():
        t = c - 1
        bits = bits_scr[t % 2]
        fb = (bits >> jnp.uint32(9)) | jnp.uint32(0x3F800000)
        u = lax.bitcast_convert_type(fb, jnp.float32) - jnp.float32(1.0)
        eps = jnp.float32(1e-10)
        g = -jnp.log(eps - jnp.log(u + eps))
        z = logits_ref[...] + g
        e = jnp.exp(z)
        e_scr[r % 2, t] = e
        m = m_scr[(t == NT - 1).astype(jnp.int32)]
        s_scr[r % 2] += jnp.where(m > jnp.float32(0.5), e, jnp.float32(0.0))

    @pl.when((r < NR) & (c < NT))
    def _hash():
        row = r * RB + lax.broadcasted_iota(jnp.int32, (RB, TW), 0)
        col = c * TW + lax.broadcasted_iota(jnp.int32, (RB, TW), 1)
        n = (row * COLS + col).astype(jnp.uint32)
        bits_scr[c % 2] = _threefry_bits(n)

    @pl.when((r > 0) & (c < NT))
    def _scale():
        out_ref[...] = e_scr[(r - 1) % 2, c] * r_scr[...]


def kernel(logits):
    def logits_idx(r, c):
        return (jnp.minimum(r, NR - 1),
                jnp.where(r < NR, jnp.where(c > 0, c - 1, 0), NT - 1))

    def out_idx(r, c):
        return (jnp.maximum(r - 1, 0),
                jnp.where(r > 0, jnp.minimum(c, NT - 1), 0))

    return pl.pallas_call(
        _kernel,
        grid=(NR + 1, NT + 1),
        in_specs=[pl.BlockSpec((RB, TW), logits_idx)],
        out_specs=pl.BlockSpec((RB, TW), out_idx),
        out_shape=jax.ShapeDtypeStruct((ROWS, COLS), jnp.float32),
        scratch_shapes=[
            pltpu.VMEM((2, NT, RB, TW), jnp.float32),
            pltpu.VMEM((2, RB, TW), jnp.uint32),
            pltpu.VMEM((2, RB, TW), jnp.float32),
            pltpu.VMEM((RB, TW), jnp.float32),
            pltpu.VMEM((2, RB, TW), jnp.float32),
        ],
    )(logits)
